# baseline (speedup 1.0000x reference)
import functools

import jax
import jax.numpy as jnp
from jax import lax
from jax.experimental import pallas as pl
from jax.experimental.pallas import tpu as pltpu
from jax.experimental.pallas import tpu_sc as plsc

N_NODES = 10000
N_EDGES = 320000
D = 128
HALF = 64
MAX_NORM = 1e6
MIN_NORM = 1e-15
EPS = 1e-5

NSUB = 16
CH = 128
NCH = 160
EPT = NCH * CH
N_PAD = 10240
ROWS_PER_SUB = N_PAD // NSUB
ZROWS = 128


def _norm(x):
    return jnp.clip(jnp.sqrt(jnp.sum(x * x, axis=-1, keepdims=True)), MIN_NORM, None)


def _artanh(x):
    x = jnp.clip(x, -1.0 + EPS, 1.0 - EPS)
    return 0.5 * jnp.log((1.0 + x) / (1.0 - x))


def _proj(x):
    maxn = 1.0 - 1e-3
    xn = _norm(x)
    return jnp.where(xn > maxn, x / xn * maxn, x)


def _mobius_add(x, y):
    x2 = jnp.sum(x * x, axis=-1, keepdims=True)
    y2 = jnp.sum(y * y, axis=-1, keepdims=True)
    xy = jnp.sum(x * y, axis=-1, keepdims=True)
    num = (1.0 + 2.0 * xy + y2) * x + (1.0 - x2) * y
    den = 1.0 + 2.0 * xy + x2 * y2
    return num / jnp.clip(den, MIN_NORM, None)


def _prologue_body(x_ref, w_ref, b_ref, o_ref):
    x = x_ref[...]
    W = w_ref[...]
    b = b_ref[...]
    for i in range(2):
        lo = HALF * i
        xi = x[:, lo:lo + HALF]
        Wi = W[lo:lo + HALF, lo:lo + HALF]
        bi = b[:, lo:lo + HALF]
        xn = _norm(xi)
        mx = lax.dot_general(xi, Wi, (((1,), (1,)), ((), ())),
                             preferred_element_type=jnp.float32)
        mxn = _norm(mx)
        res = jnp.tanh(mxn / xn * _artanh(xn)) * mx / mxn
        res = jnp.where(jnp.all(mx == 0.0, axis=-1, keepdims=True), 0.0, res)
        h = _proj(res)
        bn = _norm(bi)
        hb = _proj(jnp.tanh(bn) * bi / bn)
        h = _proj(_mobius_add(h, hb))
        hn = _norm(h)
        o_ref[i, ...] = _artanh(hn) * h / hn


def _epilogue_body(pa_ref, pb_ref, o_ref):
    outs = []
    for i, ref in enumerate((pa_ref, pb_ref)):
        p = jnp.minimum(ref[...], MAX_NORM)
        un = _norm(p)
        h = _proj(jnp.tanh(un) * p / un)
        hn = _norm(h)
        t = _artanh(hn) * h / hn
        t = jnp.minimum(jnp.maximum(t, 0.0), MAX_NORM)
        tn = _norm(t)
        outs.append(jnp.tanh(tn) * t / tn)
    o_ref[...] = jnp.concatenate(outs, axis=-1)


_ROWS_BLK = 1000


def _run_prologue(x, W, b2):
    grid = (N_NODES // _ROWS_BLK,)
    return pl.pallas_call(
        _prologue_body,
        grid=grid,
        in_specs=[
            pl.BlockSpec((_ROWS_BLK, D), lambda i: (i, 0)),
            pl.BlockSpec((D, D), lambda i: (0, 0)),
            pl.BlockSpec((1, D), lambda i: (0, 0)),
        ],
        out_specs=pl.BlockSpec((2, _ROWS_BLK, HALF), lambda i: (0, i, 0)),
        out_shape=jax.ShapeDtypeStruct((2, N_NODES, HALF), jnp.float32),
    )(x, W, b2)


def _run_epilogue(pa, pb):
    grid = (N_NODES // _ROWS_BLK,)
    return pl.pallas_call(
        _epilogue_body,
        grid=grid,
        in_specs=[
            pl.BlockSpec((_ROWS_BLK, HALF), lambda i: (i, 0)),
            pl.BlockSpec((_ROWS_BLK, HALF), lambda i: (i, 0)),
        ],
        out_specs=pl.BlockSpec((_ROWS_BLK, D), lambda i: (i, 0)),
        out_shape=jax.ShapeDtypeStruct((N_NODES, D), jnp.float32),
    )(pa, pb)


_GDN = lax.GatherDimensionNumbers(offset_dims=(), collapsed_slice_dims=(0,),
                                  start_index_map=(0,))


def _sc_agg_body(xt_hbm, src_hbm, dst_hbm, w_hbm, out_hbm,
                 src_v, dst_v, w_v, rows0_v, rows1_v, zbuf_v, acc,
                 sem0, sem1):
    cid = lax.axis_index("c")
    sid = lax.axis_index("s")

    def zrow(r, carry):
        for j in range(HALF // 16):
            zbuf_v[r, pl.ds(j * 16, 16)] = jnp.zeros((16,), jnp.float32)
        return carry
    lax.fori_loop(0, ZROWS, zrow, None)
    for i in range(ROWS_PER_SUB // ZROWS):
        pltpu.sync_copy(zbuf_v,
                        acc.at[pl.ds(sid * ROWS_PER_SUB + i * ZROWS, ZROWS)])
    plsc.subcore_barrier()

    pltpu.sync_copy(src_hbm.at[cid, sid], src_v)
    pltpu.sync_copy(dst_hbm.at[sid], dst_v)
    pltpu.sync_copy(w_hbm.at[sid], w_v)

    def scale(k, rbuf):
        def group(g, carry):
            wvec = w_v[k, pl.ds(g * 16, 16)]
            for i in range(16):
                wb = lax.gather(
                    wvec, jnp.full((16, 1), i, jnp.int32), _GDN,
                    slice_sizes=(1,),
                    mode=lax.GatherScatterMode.PROMISE_IN_BOUNDS)
                e = g * 16 + i
                for j in range(HALF // 16):
                    sl = pl.ds(j * 16, 16)
                    rbuf[e, sl] = rbuf[e, sl] * wb
            return carry
        lax.fori_loop(0, CH // 16, group, None)

    bufs = ((rows0_v, sem0), (rows1_v, sem1))
    for b in range(2):
        rbuf, sem = bufs[b]
        pltpu.make_async_copy(xt_hbm.at[src_v.at[b]], rbuf, sem).start()

    def pair(i, carry):
        for b in range(2):
            kk = 2 * i + b
            rbuf, sem = bufs[b]
            pltpu.make_async_copy(xt_hbm.at[src_v.at[kk]], rbuf, sem).wait()
            scale(kk, rbuf)
            pltpu.sync_copy(rbuf, acc.at[dst_v.at[kk]], add=True)

            @pl.when(kk + 2 < NCH)
            def _prefetch():
                pltpu.make_async_copy(
                    xt_hbm.at[src_v.at[kk + 2]], rbuf, sem).start()
        return carry
    lax.fori_loop(0, NCH // 2, pair, None)
    plsc.subcore_barrier()

    for i in range(ROWS_PER_SUB // ZROWS):
        r0 = sid * ROWS_PER_SUB + i * ZROWS
        pltpu.sync_copy(acc.at[pl.ds(r0, ZROWS)], out_hbm.at[cid, pl.ds(r0, ZROWS)])


def _run_sc_agg(xt2, src, dst, w):
    mesh = plsc.VectorSubcoreMesh(core_axis_name="c", subcore_axis_name="s")
    k = functools.partial(
        pl.kernel,
        mesh=mesh,
        compiler_params=pltpu.CompilerParams(use_tc_tiling_on_sc=False),
        out_type=jax.ShapeDtypeStruct((2, N_PAD, HALF), jnp.float32),
        scratch_types=[
            pltpu.VMEM((NCH, CH), jnp.int32),
            pltpu.VMEM((NCH, CH), jnp.int32),
            pltpu.VMEM((NCH, CH), jnp.float32),
            pltpu.VMEM((CH, HALF), jnp.float32),
            pltpu.VMEM((CH, HALF), jnp.float32),
            pltpu.VMEM((ZROWS, HALF), jnp.float32),
            pltpu.VMEM_SHARED((N_PAD, HALF), jnp.float32),
            pltpu.SemaphoreType.DMA,
            pltpu.SemaphoreType.DMA,
        ],
    )(_sc_agg_body)
    return k(xt2, src, dst, w)


def kernel(x, edge_index, edge_weight, W, b):
    x = x.astype(jnp.float32)
    W = W.astype(jnp.float32)
    b2 = b.astype(jnp.float32).reshape(1, D)

    src0 = edge_index[0].astype(jnp.int32).reshape(NSUB, 20000)
    dst0 = edge_index[1].astype(jnp.int32).reshape(NSUB, 20000)
    w0 = edge_weight.astype(jnp.float32).reshape(NSUB, 20000)
    pad = EPT - 20000
    src0 = jnp.pad(src0, ((0, 0), (0, pad))).reshape(NSUB, NCH, CH)
    dst0 = jnp.pad(dst0, ((0, 0), (0, pad)),
                   constant_values=N_PAD - 1).reshape(NSUB, NCH, CH)
    w0 = jnp.pad(w0, ((0, 0), (0, pad))).reshape(NSUB, NCH, CH)
    src2 = jnp.stack([src0, src0 + N_NODES])

    xt2 = _run_prologue(x, W, b2)
    xt_tab = xt2.reshape(2 * N_NODES, HALF)
    parts = _run_sc_agg(xt_tab, src2, dst0, w0)
    return _run_epilogue(parts[0, :N_NODES], parts[1, :N_NODES])


# speedup vs baseline: 1.5730x; 1.5730x over previous
import functools

import jax
import jax.numpy as jnp
from jax import lax
from jax.experimental import pallas as pl
from jax.experimental.pallas import tpu as pltpu
from jax.experimental.pallas import tpu_sc as plsc

N_NODES = 10000
N_EDGES = 320000
D = 128
HALF = 64
MAX_NORM = 1e6
MIN_NORM = 1e-15
EPS = 1e-5

NSUB = 16
CH = 128
NCH = 160
EPT = NCH * CH
N_PAD = 10240
ROWS_PER_SUB = N_PAD // NSUB
ZROWS = 128


def _norm(x):
    return jnp.clip(jnp.sqrt(jnp.sum(x * x, axis=-1, keepdims=True)), MIN_NORM, None)


def _artanh(x):
    x = jnp.clip(x, -1.0 + EPS, 1.0 - EPS)
    return 0.5 * jnp.log((1.0 + x) / (1.0 - x))


def _proj(x):
    maxn = 1.0 - 1e-3
    xn = _norm(x)
    return jnp.where(xn > maxn, x / xn * maxn, x)


def _mobius_add(x, y):
    x2 = jnp.sum(x * x, axis=-1, keepdims=True)
    y2 = jnp.sum(y * y, axis=-1, keepdims=True)
    xy = jnp.sum(x * y, axis=-1, keepdims=True)
    num = (1.0 + 2.0 * xy + y2) * x + (1.0 - x2) * y
    den = 1.0 + 2.0 * xy + x2 * y2
    return num / jnp.clip(den, MIN_NORM, None)


def _prologue_body(x_ref, w_ref, b_ref, o_ref):
    x = x_ref[...]
    W = w_ref[...]
    b = b_ref[...]
    for i in range(2):
        lo = HALF * i
        xi = x[:, lo:lo + HALF]
        Wi = W[lo:lo + HALF, lo:lo + HALF]
        bi = b[:, lo:lo + HALF]
        xn = _norm(xi)
        mx = lax.dot_general(xi, Wi, (((1,), (1,)), ((), ())),
                             preferred_element_type=jnp.float32)
        mxn = _norm(mx)
        res = jnp.tanh(mxn / xn * _artanh(xn)) * mx / mxn
        res = jnp.where(jnp.all(mx == 0.0, axis=-1, keepdims=True), 0.0, res)
        h = _proj(res)
        bn = _norm(bi)
        hb = _proj(jnp.tanh(bn) * bi / bn)
        h = _proj(_mobius_add(h, hb))
        hn = _norm(h)
        o_ref[i, ...] = _artanh(hn) * h / hn


def _epilogue_body(pa_ref, pb_ref, o_ref):
    outs = []
    for i, ref in enumerate((pa_ref, pb_ref)):
        p = jnp.minimum(ref[...], MAX_NORM)
        un = _norm(p)
        h = _proj(jnp.tanh(un) * p / un)
        hn = _norm(h)
        t = _artanh(hn) * h / hn
        t = jnp.minimum(jnp.maximum(t, 0.0), MAX_NORM)
        tn = _norm(t)
        outs.append(jnp.tanh(tn) * t / tn)
    o_ref[...] = jnp.concatenate(outs, axis=-1)


_ROWS_BLK = 1000


def _run_prologue(x, W, b2):
    grid = (N_NODES // _ROWS_BLK,)
    return pl.pallas_call(
        _prologue_body,
        grid=grid,
        in_specs=[
            pl.BlockSpec((_ROWS_BLK, D), lambda i: (i, 0)),
            pl.BlockSpec((D, D), lambda i: (0, 0)),
            pl.BlockSpec((1, D), lambda i: (0, 0)),
        ],
        out_specs=pl.BlockSpec((2, _ROWS_BLK, HALF), lambda i: (0, i, 0)),
        out_shape=jax.ShapeDtypeStruct((2, N_NODES, HALF), jnp.float32),
    )(x, W, b2)


def _run_epilogue(pa, pb):
    grid = (N_NODES // _ROWS_BLK,)
    return pl.pallas_call(
        _epilogue_body,
        grid=grid,
        in_specs=[
            pl.BlockSpec((_ROWS_BLK, HALF), lambda i: (i, 0)),
            pl.BlockSpec((_ROWS_BLK, HALF), lambda i: (i, 0)),
        ],
        out_specs=pl.BlockSpec((_ROWS_BLK, D), lambda i: (i, 0)),
        out_shape=jax.ShapeDtypeStruct((N_NODES, D), jnp.float32),
    )(pa, pb)


_GDN = lax.GatherDimensionNumbers(offset_dims=(), collapsed_slice_dims=(0,),
                                  start_index_map=(0,))


def _sc_agg_body(xt_hbm, src_hbm, dst_hbm, w_hbm, out_hbm,
                 src_v, dst_v, w_v, rows0_v, rows1_v, zbuf_v, acc,
                 sem0, sem1):
    cid = lax.axis_index("c")
    sid = lax.axis_index("s")

    def zrow(r, carry):
        for j in range(HALF // 16):
            zbuf_v[r, pl.ds(j * 16, 16)] = jnp.zeros((16,), jnp.float32)
        return carry
    lax.fori_loop(0, ZROWS, zrow, None)
    for i in range(ROWS_PER_SUB // ZROWS):
        pltpu.sync_copy(zbuf_v,
                        acc.at[pl.ds(sid * ROWS_PER_SUB + i * ZROWS, ZROWS)])
    plsc.subcore_barrier()

    pltpu.sync_copy(src_hbm.at[cid, sid], src_v)
    pltpu.sync_copy(dst_hbm.at[sid], dst_v)
    pltpu.sync_copy(w_hbm.at[sid], w_v)

    def scale(k, rbuf):
        for g in range(CH // 16):
            wvec = w_v[k, pl.ds(g * 16, 16)]
            for i in range(16):
                wb = lax.gather(
                    wvec, jnp.full((16, 1), i, jnp.int32), _GDN,
                    slice_sizes=(1,),
                    mode=lax.GatherScatterMode.PROMISE_IN_BOUNDS)
                e = g * 16 + i
                for j in range(HALF // 16):
                    sl = pl.ds(j * 16, 16)
                    rbuf[e, sl] = rbuf[e, sl] * wb

    bufs = ((rows0_v, sem0), (rows1_v, sem1))
    for b in range(2):
        rbuf, sem = bufs[b]
        pltpu.make_async_copy(xt_hbm.at[src_v.at[b]], rbuf, sem).start()

    def pair(i, carry):
        for b in range(2):
            kk = 2 * i + b
            rbuf, sem = bufs[b]
            pltpu.make_async_copy(xt_hbm.at[src_v.at[kk]], rbuf, sem).wait()
            scale(kk, rbuf)
            pltpu.sync_copy(rbuf, acc.at[dst_v.at[kk]], add=True)

            @pl.when(kk + 2 < NCH)
            def _prefetch():
                pltpu.make_async_copy(
                    xt_hbm.at[src_v.at[kk + 2]], rbuf, sem).start()
        return carry
    lax.fori_loop(0, NCH // 2, pair, None)
    plsc.subcore_barrier()

    for i in range(ROWS_PER_SUB // ZROWS):
        r0 = sid * ROWS_PER_SUB + i * ZROWS
        pltpu.sync_copy(acc.at[pl.ds(r0, ZROWS)], out_hbm.at[cid, pl.ds(r0, ZROWS)])


def _run_sc_agg(xt2, src, dst, w):
    mesh = plsc.VectorSubcoreMesh(core_axis_name="c", subcore_axis_name="s")
    k = functools.partial(
        pl.kernel,
        mesh=mesh,
        compiler_params=pltpu.CompilerParams(use_tc_tiling_on_sc=False),
        out_type=jax.ShapeDtypeStruct((2, N_PAD, HALF), jnp.float32),
        scratch_types=[
            pltpu.VMEM((NCH, CH), jnp.int32),
            pltpu.VMEM((NCH, CH), jnp.int32),
            pltpu.VMEM((NCH, CH), jnp.float32),
            pltpu.VMEM((CH, HALF), jnp.float32),
            pltpu.VMEM((CH, HALF), jnp.float32),
            pltpu.VMEM((ZROWS, HALF), jnp.float32),
            pltpu.VMEM_SHARED((N_PAD, HALF), jnp.float32),
            pltpu.SemaphoreType.DMA,
            pltpu.SemaphoreType.DMA,
        ],
    )(_sc_agg_body)
    return k(xt2, src, dst, w)


def kernel(x, edge_index, edge_weight, W, b):
    x = x.astype(jnp.float32)
    W = W.astype(jnp.float32)
    b2 = b.astype(jnp.float32).reshape(1, D)

    src0 = edge_index[0].astype(jnp.int32).reshape(NSUB, 20000)
    dst0 = edge_index[1].astype(jnp.int32).reshape(NSUB, 20000)
    w0 = edge_weight.astype(jnp.float32).reshape(NSUB, 20000)
    pad = EPT - 20000
    src0 = jnp.pad(src0, ((0, 0), (0, pad))).reshape(NSUB, NCH, CH)
    dst0 = jnp.pad(dst0, ((0, 0), (0, pad)),
                   constant_values=N_PAD - 1).reshape(NSUB, NCH, CH)
    w0 = jnp.pad(w0, ((0, 0), (0, pad))).reshape(NSUB, NCH, CH)
    src2 = jnp.stack([src0, src0 + N_NODES])

    xt2 = _run_prologue(x, W, b2)
    xt_tab = xt2.reshape(2 * N_NODES, HALF)
    parts = _run_sc_agg(xt_tab, src2, dst0, w0)
    return _run_epilogue(parts[0, :N_NODES], parts[1, :N_NODES])


# speedup vs baseline: 2.1091x; 1.3408x over previous
import functools

import jax
import jax.numpy as jnp
from jax import lax
from jax.experimental import pallas as pl
from jax.experimental.pallas import tpu as pltpu
from jax.experimental.pallas import tpu_sc as plsc

N_NODES = 10000
N_EDGES = 320000
D = 128
HALF = 64
MAX_NORM = 1e6
MIN_NORM = 1e-15
EPS = 1e-5

NSUB = 16
CH = 128
NCH = 160
EPT = NCH * CH
N_PAD = 10240
ROWS_PER_SUB = N_PAD // NSUB
ZROWS = 128


def _norm(x):
    return jnp.clip(jnp.sqrt(jnp.sum(x * x, axis=-1, keepdims=True)), MIN_NORM, None)


def _artanh(x):
    x = jnp.clip(x, -1.0 + EPS, 1.0 - EPS)
    return 0.5 * jnp.log((1.0 + x) / (1.0 - x))


def _proj(x):
    maxn = 1.0 - 1e-3
    xn = _norm(x)
    return jnp.where(xn > maxn, x / xn * maxn, x)


def _mobius_add(x, y):
    x2 = jnp.sum(x * x, axis=-1, keepdims=True)
    y2 = jnp.sum(y * y, axis=-1, keepdims=True)
    xy = jnp.sum(x * y, axis=-1, keepdims=True)
    num = (1.0 + 2.0 * xy + y2) * x + (1.0 - x2) * y
    den = 1.0 + 2.0 * xy + x2 * y2
    return num / jnp.clip(den, MIN_NORM, None)


def _prologue_body(x_ref, w_ref, b_ref, o_ref):
    x = x_ref[...]
    W = w_ref[...]
    b = b_ref[...]
    for i in range(2):
        lo = HALF * i
        xi = x[:, lo:lo + HALF]
        Wi = W[lo:lo + HALF, lo:lo + HALF]
        bi = b[:, lo:lo + HALF]
        xn = _norm(xi)
        mx = lax.dot_general(xi, Wi, (((1,), (1,)), ((), ())),
                             preferred_element_type=jnp.float32)
        mxn = _norm(mx)
        res = jnp.tanh(mxn / xn * _artanh(xn)) * mx / mxn
        res = jnp.where(jnp.all(mx == 0.0, axis=-1, keepdims=True), 0.0, res)
        h = _proj(res)
        bn = _norm(bi)
        hb = _proj(jnp.tanh(bn) * bi / bn)
        h = _proj(_mobius_add(h, hb))
        hn = _norm(h)
        o_ref[i, ...] = _artanh(hn) * h / hn


def _epilogue_body(pa_ref, pb_ref, o_ref):
    outs = []
    for i, ref in enumerate((pa_ref, pb_ref)):
        p = jnp.minimum(ref[...], MAX_NORM)
        un = _norm(p)
        h = _proj(jnp.tanh(un) * p / un)
        hn = _norm(h)
        t = _artanh(hn) * h / hn
        t = jnp.minimum(jnp.maximum(t, 0.0), MAX_NORM)
        tn = _norm(t)
        outs.append(jnp.tanh(tn) * t / tn)
    o_ref[...] = jnp.concatenate(outs, axis=-1)


_ROWS_BLK = 1000


def _run_prologue(x, W, b2):
    grid = (N_NODES // _ROWS_BLK,)
    return pl.pallas_call(
        _prologue_body,
        grid=grid,
        in_specs=[
            pl.BlockSpec((_ROWS_BLK, D), lambda i: (i, 0)),
            pl.BlockSpec((D, D), lambda i: (0, 0)),
            pl.BlockSpec((1, D), lambda i: (0, 0)),
        ],
        out_specs=pl.BlockSpec((2, _ROWS_BLK, HALF), lambda i: (0, i, 0)),
        out_shape=jax.ShapeDtypeStruct((2, N_NODES, HALF), jnp.float32),
    )(x, W, b2)


def _run_epilogue(pa, pb):
    grid = (N_NODES // _ROWS_BLK,)
    return pl.pallas_call(
        _epilogue_body,
        grid=grid,
        in_specs=[
            pl.BlockSpec((_ROWS_BLK, HALF), lambda i: (i, 0)),
            pl.BlockSpec((_ROWS_BLK, HALF), lambda i: (i, 0)),
        ],
        out_specs=pl.BlockSpec((_ROWS_BLK, D), lambda i: (i, 0)),
        out_shape=jax.ShapeDtypeStruct((N_NODES, D), jnp.float32),
    )(pa, pb)


_GDN = lax.GatherDimensionNumbers(offset_dims=(), collapsed_slice_dims=(0,),
                                  start_index_map=(0,))

BLK = 32
TAB_STRIPE = N_NODES // NSUB


def _sc_agg_body(xt_hbm, src_hbm, dst_hbm, w_hbm, out_hbm,
                 srcb_v, dstb_v, wb_v, rows0_v, rows1_v, zbuf_v, tab, acc,
                 sem0, sem1):
    cid = lax.axis_index("c")
    sid = lax.axis_index("s")

    def zrow(r, carry):
        for j in range(HALF // 16):
            zbuf_v[r, pl.ds(j * 16, 16)] = jnp.zeros((16,), jnp.float32)
        return carry
    lax.fori_loop(0, ZROWS, zrow, None)
    for i in range(ROWS_PER_SUB // ZROWS):
        pltpu.sync_copy(zbuf_v,
                        acc.at[pl.ds(sid * ROWS_PER_SUB + i * ZROWS, ZROWS)])
    t0 = sid * TAB_STRIPE
    pltpu.sync_copy(xt_hbm.at[cid, pl.ds(t0, TAB_STRIPE)],
                    tab.at[pl.ds(t0, TAB_STRIPE)])
    plsc.subcore_barrier()

    def scale(k, rbuf):
        for g in range(CH // 16):
            wvec = wb_v[k, pl.ds(g * 16, 16)]
            for i in range(16):
                wb = lax.gather(
                    wvec, jnp.full((16, 1), i, jnp.int32), _GDN,
                    slice_sizes=(1,),
                    mode=lax.GatherScatterMode.PROMISE_IN_BOUNDS)
                e = g * 16 + i
                for j in range(HALF // 16):
                    sl = pl.ds(j * 16, 16)
                    rbuf[e, sl] = rbuf[e, sl] * wb

    bufs = ((rows0_v, sem0), (rows1_v, sem1))

    def block(blk, carry):
        b0 = blk * BLK
        pltpu.sync_copy(src_hbm.at[sid, pl.ds(b0, BLK)], srcb_v)
        pltpu.sync_copy(dst_hbm.at[sid, pl.ds(b0, BLK)], dstb_v)
        pltpu.sync_copy(w_hbm.at[sid, pl.ds(b0, BLK)], wb_v)
        for b in range(2):
            rbuf, sem = bufs[b]
            pltpu.make_async_copy(tab.at[srcb_v.at[b]], rbuf, sem).start()

        def pair(i, carry2):
            for b in range(2):
                kk = 2 * i + b
                rbuf, sem = bufs[b]
                pltpu.make_async_copy(tab.at[srcb_v.at[kk]], rbuf, sem).wait()
                scale(kk, rbuf)
                pltpu.sync_copy(rbuf, acc.at[dstb_v.at[kk]], add=True)

                @pl.when(kk + 2 < BLK)
                def _prefetch():
                    pltpu.make_async_copy(
                        tab.at[srcb_v.at[kk + 2]], rbuf, sem).start()
            return carry2
        lax.fori_loop(0, BLK // 2, pair, None)
        return carry
    lax.fori_loop(0, NCH // BLK, block, None)
    plsc.subcore_barrier()

    for i in range(ROWS_PER_SUB // ZROWS):
        r0 = sid * ROWS_PER_SUB + i * ZROWS
        pltpu.sync_copy(acc.at[pl.ds(r0, ZROWS)], out_hbm.at[cid, pl.ds(r0, ZROWS)])


def _run_sc_agg(xt2, src, dst, w):
    mesh = plsc.VectorSubcoreMesh(core_axis_name="c", subcore_axis_name="s")
    k = functools.partial(
        pl.kernel,
        mesh=mesh,
        compiler_params=pltpu.CompilerParams(use_tc_tiling_on_sc=False),
        out_type=jax.ShapeDtypeStruct((2, N_PAD, HALF), jnp.float32),
        scratch_types=[
            pltpu.VMEM((BLK, CH), jnp.int32),
            pltpu.VMEM((BLK, CH), jnp.int32),
            pltpu.VMEM((BLK, CH), jnp.float32),
            pltpu.VMEM((CH, HALF), jnp.float32),
            pltpu.VMEM((CH, HALF), jnp.float32),
            pltpu.VMEM((ZROWS, HALF), jnp.float32),
            pltpu.VMEM_SHARED((N_NODES, HALF), jnp.float32),
            pltpu.VMEM_SHARED((N_PAD, HALF), jnp.float32),
            pltpu.SemaphoreType.DMA,
            pltpu.SemaphoreType.DMA,
        ],
    )(_sc_agg_body)
    return k(xt2, src, dst, w)


def kernel(x, edge_index, edge_weight, W, b):
    x = x.astype(jnp.float32)
    W = W.astype(jnp.float32)
    b2 = b.astype(jnp.float32).reshape(1, D)

    src0 = edge_index[0].astype(jnp.int32).reshape(NSUB, 20000)
    dst0 = edge_index[1].astype(jnp.int32).reshape(NSUB, 20000)
    w0 = edge_weight.astype(jnp.float32).reshape(NSUB, 20000)
    pad = EPT - 20000
    src0 = jnp.pad(src0, ((0, 0), (0, pad))).reshape(NSUB, NCH, CH)
    dst0 = jnp.pad(dst0, ((0, 0), (0, pad)),
                   constant_values=N_PAD - 1).reshape(NSUB, NCH, CH)
    w0 = jnp.pad(w0, ((0, 0), (0, pad))).reshape(NSUB, NCH, CH)

    xt2 = _run_prologue(x, W, b2)
    parts = _run_sc_agg(xt2, src0, dst0, w0)
    return _run_epilogue(parts[0, :N_NODES], parts[1, :N_NODES])


# speedup vs baseline: 2.3036x; 1.0922x over previous
import functools

import jax
import jax.numpy as jnp
from jax import lax
from jax.experimental import pallas as pl
from jax.experimental.pallas import tpu as pltpu
from jax.experimental.pallas import tpu_sc as plsc

N_NODES = 10000
N_EDGES = 320000
D = 128
HALF = 64
MAX_NORM = 1e6
MIN_NORM = 1e-15
EPS = 1e-5

NSUB = 16
CH = 128
NCH = 160
EPT = NCH * CH
N_PAD = 10240
ROWS_PER_SUB = N_PAD // NSUB
ZROWS = 128


def _norm(x):
    return jnp.clip(jnp.sqrt(jnp.sum(x * x, axis=-1, keepdims=True)), MIN_NORM, None)


def _artanh(x):
    x = jnp.clip(x, -1.0 + EPS, 1.0 - EPS)
    return 0.5 * jnp.log((1.0 + x) / (1.0 - x))


def _proj(x):
    maxn = 1.0 - 1e-3
    xn = _norm(x)
    return jnp.where(xn > maxn, x / xn * maxn, x)


def _mobius_add(x, y):
    x2 = jnp.sum(x * x, axis=-1, keepdims=True)
    y2 = jnp.sum(y * y, axis=-1, keepdims=True)
    xy = jnp.sum(x * y, axis=-1, keepdims=True)
    num = (1.0 + 2.0 * xy + y2) * x + (1.0 - x2) * y
    den = 1.0 + 2.0 * xy + x2 * y2
    return num / jnp.clip(den, MIN_NORM, None)


def _prologue_body(x_ref, w_ref, b_ref, o_ref):
    x = x_ref[...]
    W = w_ref[...]
    b = b_ref[...]
    for i in range(2):
        lo = HALF * i
        xi = x[:, lo:lo + HALF]
        Wi = W[lo:lo + HALF, lo:lo + HALF]
        bi = b[:, lo:lo + HALF]
        xn = _norm(xi)
        mx = lax.dot_general(xi, Wi, (((1,), (1,)), ((), ())),
                             preferred_element_type=jnp.float32)
        mxn = _norm(mx)
        res = jnp.tanh(mxn / xn * _artanh(xn)) * mx / mxn
        res = jnp.where(jnp.all(mx == 0.0, axis=-1, keepdims=True), 0.0, res)
        h = _proj(res)
        bn = _norm(bi)
        hb = _proj(jnp.tanh(bn) * bi / bn)
        h = _proj(_mobius_add(h, hb))
        hn = _norm(h)
        o_ref[i, ...] = _artanh(hn) * h / hn


def _epilogue_body(pa_ref, pb_ref, o_ref):
    outs = []
    for i, ref in enumerate((pa_ref, pb_ref)):
        p = jnp.minimum(ref[...], MAX_NORM)
        un = _norm(p)
        h = _proj(jnp.tanh(un) * p / un)
        hn = _norm(h)
        t = _artanh(hn) * h / hn
        t = jnp.minimum(jnp.maximum(t, 0.0), MAX_NORM)
        tn = _norm(t)
        outs.append(jnp.tanh(tn) * t / tn)
    o_ref[...] = jnp.concatenate(outs, axis=-1)


_ROWS_BLK = 1000


def _run_prologue(x, W, b2):
    grid = (N_NODES // _ROWS_BLK,)
    return pl.pallas_call(
        _prologue_body,
        grid=grid,
        in_specs=[
            pl.BlockSpec((_ROWS_BLK, D), lambda i: (i, 0)),
            pl.BlockSpec((D, D), lambda i: (0, 0)),
            pl.BlockSpec((1, D), lambda i: (0, 0)),
        ],
        out_specs=pl.BlockSpec((2, _ROWS_BLK, HALF), lambda i: (0, i, 0)),
        out_shape=jax.ShapeDtypeStruct((2, N_NODES, HALF), jnp.float32),
    )(x, W, b2)


def _run_epilogue(pa, pb):
    grid = (N_NODES // _ROWS_BLK,)
    return pl.pallas_call(
        _epilogue_body,
        grid=grid,
        in_specs=[
            pl.BlockSpec((_ROWS_BLK, HALF), lambda i: (i, 0)),
            pl.BlockSpec((_ROWS_BLK, HALF), lambda i: (i, 0)),
        ],
        out_specs=pl.BlockSpec((_ROWS_BLK, D), lambda i: (i, 0)),
        out_shape=jax.ShapeDtypeStruct((N_NODES, D), jnp.float32),
    )(pa, pb)


_GDN = lax.GatherDimensionNumbers(offset_dims=(), collapsed_slice_dims=(0,),
                                  start_index_map=(0,))

BLK = 32
TAB_STRIPE = N_NODES // NSUB


def _sc_agg_body(xt_hbm, src_hbm, dst_hbm, w_hbm, out_hbm,
                 srcb_v, dstb_v, wb_v, rows0_v, rows1_v, sbuf0_v, sbuf1_v,
                 tab, acc, sem0, sem1, ssem0, ssem1):
    cid = lax.axis_index("c")
    sid = lax.axis_index("s")

    def zrow(r, carry):
        for j in range(HALF // 16):
            rows0_v[r, pl.ds(j * 16, 16)] = jnp.zeros((16,), jnp.float32)
        return carry
    lax.fori_loop(0, ZROWS, zrow, None)
    for i in range(ROWS_PER_SUB // ZROWS):
        pltpu.sync_copy(rows0_v,
                        acc.at[pl.ds(sid * ROWS_PER_SUB + i * ZROWS, ZROWS)])
    t0 = sid * TAB_STRIPE
    pltpu.sync_copy(xt_hbm.at[cid, pl.ds(t0, TAB_STRIPE)],
                    tab.at[pl.ds(t0, TAB_STRIPE)])
    plsc.subcore_barrier()

    def scale(k, rbuf, sbuf):
        for g in range(CH // 16):
            wvec = wb_v[k, pl.ds(g * 16, 16)]
            for i in range(16):
                wb = lax.gather(
                    wvec, jnp.full((16, 1), i, jnp.int32), _GDN,
                    slice_sizes=(1,),
                    mode=lax.GatherScatterMode.PROMISE_IN_BOUNDS)
                e = g * 16 + i
                for j in range(HALF // 16):
                    sl = pl.ds(j * 16, 16)
                    sbuf[e, sl] = rbuf[e, sl] * wb

    gbufs = ((rows0_v, sem0), (rows1_v, sem1))
    sbufs = ((sbuf0_v, ssem0), (sbuf1_v, ssem1))

    def block(blk, carry):
        b0 = blk * BLK
        pltpu.sync_copy(src_hbm.at[sid, pl.ds(b0, BLK)], srcb_v)
        pltpu.sync_copy(dst_hbm.at[sid, pl.ds(b0, BLK)], dstb_v)
        pltpu.sync_copy(w_hbm.at[sid, pl.ds(b0, BLK)], wb_v)
        for b in range(2):
            rbuf, sem = gbufs[b]
            pltpu.make_async_copy(tab.at[srcb_v.at[b]], rbuf, sem).start()

        def pair(i, carry2):
            for b in range(2):
                kk = 2 * i + b
                rbuf, gsem = gbufs[b]
                sbuf, ssem = sbufs[b]

                @pl.when(kk >= 2)
                def _drain():
                    pltpu.make_async_copy(
                        sbuf, acc.at[dstb_v.at[kk]], ssem).wait()
                pltpu.make_async_copy(tab.at[srcb_v.at[kk]], rbuf, gsem).wait()
                scale(kk, rbuf, sbuf)
                pltpu.async_copy(sbuf, acc.at[dstb_v.at[kk]], ssem, add=True)

                @pl.when(kk + 2 < BLK)
                def _prefetch():
                    pltpu.make_async_copy(
                        tab.at[srcb_v.at[kk + 2]], rbuf, gsem).start()
            return carry2
        lax.fori_loop(0, BLK // 2, pair, None)
        for b in range(2):
            sbuf, ssem = sbufs[b]
            pltpu.make_async_copy(sbuf, acc.at[dstb_v.at[BLK - 2 + b]],
                                  ssem).wait()
        return carry
    lax.fori_loop(0, NCH // BLK, block, None)
    plsc.subcore_barrier()

    for i in range(ROWS_PER_SUB // ZROWS):
        r0 = sid * ROWS_PER_SUB + i * ZROWS
        pltpu.sync_copy(acc.at[pl.ds(r0, ZROWS)], out_hbm.at[cid, pl.ds(r0, ZROWS)])


def _run_sc_agg(xt2, src, dst, w):
    mesh = plsc.VectorSubcoreMesh(core_axis_name="c", subcore_axis_name="s")
    k = functools.partial(
        pl.kernel,
        mesh=mesh,
        compiler_params=pltpu.CompilerParams(use_tc_tiling_on_sc=False),
        out_type=jax.ShapeDtypeStruct((2, N_PAD, HALF), jnp.float32),
        scratch_types=[
            pltpu.VMEM((BLK, CH), jnp.int32),
            pltpu.VMEM((BLK, CH), jnp.int32),
            pltpu.VMEM((BLK, CH), jnp.float32),
            pltpu.VMEM((CH, HALF), jnp.float32),
            pltpu.VMEM((CH, HALF), jnp.float32),
            pltpu.VMEM((CH, HALF), jnp.float32),
            pltpu.VMEM((CH, HALF), jnp.float32),
            pltpu.VMEM_SHARED((N_NODES, HALF), jnp.float32),
            pltpu.VMEM_SHARED((N_PAD, HALF), jnp.float32),
            pltpu.SemaphoreType.DMA,
            pltpu.SemaphoreType.DMA,
            pltpu.SemaphoreType.DMA,
            pltpu.SemaphoreType.DMA,
        ],
    )(_sc_agg_body)
    return k(xt2, src, dst, w)


def kernel(x, edge_index, edge_weight, W, b):
    x = x.astype(jnp.float32)
    W = W.astype(jnp.float32)
    b2 = b.astype(jnp.float32).reshape(1, D)

    src0 = edge_index[0].astype(jnp.int32).reshape(NSUB, 20000)
    dst0 = edge_index[1].astype(jnp.int32).reshape(NSUB, 20000)
    w0 = edge_weight.astype(jnp.float32).reshape(NSUB, 20000)
    pad = EPT - 20000
    src0 = jnp.pad(src0, ((0, 0), (0, pad))).reshape(NSUB, NCH, CH)
    dst0 = jnp.pad(dst0, ((0, 0), (0, pad)),
                   constant_values=N_PAD - 1).reshape(NSUB, NCH, CH)
    w0 = jnp.pad(w0, ((0, 0), (0, pad))).reshape(NSUB, NCH, CH)

    xt2 = _run_prologue(x, W, b2)
    parts = _run_sc_agg(xt2, src0, dst0, w0)
    return _run_epilogue(parts[0, :N_NODES], parts[1, :N_NODES])


# speedup vs baseline: 2.3100x; 1.0028x over previous
import functools

import jax
import jax.numpy as jnp
from jax import lax
from jax.experimental import pallas as pl
from jax.experimental.pallas import tpu as pltpu
from jax.experimental.pallas import tpu_sc as plsc

N_NODES = 10000
N_EDGES = 320000
D = 128
HALF = 64
MAX_NORM = 1e6
MIN_NORM = 1e-15
EPS = 1e-5

NSUB = 16
CH = 128
NCH = 160
EPT = NCH * CH
N_PAD = 10240
ROWS_PER_SUB = N_PAD // NSUB
ZROWS = 128


def _norm(x):
    return jnp.clip(jnp.sqrt(jnp.sum(x * x, axis=-1, keepdims=True)), MIN_NORM, None)


def _artanh(x):
    x = jnp.clip(x, -1.0 + EPS, 1.0 - EPS)
    return 0.5 * jnp.log((1.0 + x) / (1.0 - x))


def _proj(x):
    maxn = 1.0 - 1e-3
    xn = _norm(x)
    return jnp.where(xn > maxn, x / xn * maxn, x)


def _mobius_add(x, y):
    x2 = jnp.sum(x * x, axis=-1, keepdims=True)
    y2 = jnp.sum(y * y, axis=-1, keepdims=True)
    xy = jnp.sum(x * y, axis=-1, keepdims=True)
    num = (1.0 + 2.0 * xy + y2) * x + (1.0 - x2) * y
    den = 1.0 + 2.0 * xy + x2 * y2
    return num / jnp.clip(den, MIN_NORM, None)


def _prologue_body(x_ref, w_ref, b_ref, o_ref):
    x = x_ref[...]
    W = w_ref[...]
    b = b_ref[...]
    for i in range(2):
        lo = HALF * i
        xi = x[:, lo:lo + HALF]
        Wi = W[lo:lo + HALF, lo:lo + HALF]
        bi = b[:, lo:lo + HALF]
        xn = _norm(xi)
        mx = lax.dot_general(xi, Wi, (((1,), (1,)), ((), ())),
                             preferred_element_type=jnp.float32)
        mxn = _norm(mx)
        res = jnp.tanh(mxn / xn * _artanh(xn)) * mx / mxn
        res = jnp.where(jnp.all(mx == 0.0, axis=-1, keepdims=True), 0.0, res)
        h = _proj(res)
        bn = _norm(bi)
        hb = _proj(jnp.tanh(bn) * bi / bn)
        h = _proj(_mobius_add(h, hb))
        hn = _norm(h)
        o_ref[i, ...] = _artanh(hn) * h / hn


def _epilogue_body(pa_ref, pb_ref, o_ref):
    outs = []
    for i, ref in enumerate((pa_ref, pb_ref)):
        p = jnp.minimum(ref[...], MAX_NORM)
        un = _norm(p)
        h = _proj(jnp.tanh(un) * p / un)
        hn = _norm(h)
        t = _artanh(hn) * h / hn
        t = jnp.minimum(jnp.maximum(t, 0.0), MAX_NORM)
        tn = _norm(t)
        outs.append(jnp.tanh(tn) * t / tn)
    o_ref[...] = jnp.concatenate(outs, axis=-1)


_ROWS_BLK = 2000


def _run_prologue(x, W, b2):
    grid = (N_NODES // _ROWS_BLK,)
    return pl.pallas_call(
        _prologue_body,
        grid=grid,
        in_specs=[
            pl.BlockSpec((_ROWS_BLK, D), lambda i: (i, 0)),
            pl.BlockSpec((D, D), lambda i: (0, 0)),
            pl.BlockSpec((1, D), lambda i: (0, 0)),
        ],
        out_specs=pl.BlockSpec((2, _ROWS_BLK, HALF), lambda i: (0, i, 0)),
        out_shape=jax.ShapeDtypeStruct((2, N_NODES, HALF), jnp.float32),
    )(x, W, b2)


def _run_epilogue(pa, pb):
    grid = (N_NODES // _ROWS_BLK,)
    return pl.pallas_call(
        _epilogue_body,
        grid=grid,
        in_specs=[
            pl.BlockSpec((_ROWS_BLK, HALF), lambda i: (i, 0)),
            pl.BlockSpec((_ROWS_BLK, HALF), lambda i: (i, 0)),
        ],
        out_specs=pl.BlockSpec((_ROWS_BLK, D), lambda i: (i, 0)),
        out_shape=jax.ShapeDtypeStruct((N_NODES, D), jnp.float32),
    )(pa, pb)


_GDN = lax.GatherDimensionNumbers(offset_dims=(), collapsed_slice_dims=(0,),
                                  start_index_map=(0,))

BLK = 32
TAB_STRIPE = N_NODES // NSUB


def _sc_agg_body(xt_hbm, src_hbm, dst_hbm, w_hbm, out_hbm,
                 srcb_v, dstb_v, wb_v, rows0_v, rows1_v, sbuf0_v, sbuf1_v,
                 tab, acc, sem0, sem1, ssem0, ssem1):
    cid = lax.axis_index("c")
    sid = lax.axis_index("s")

    def zrow(r, carry):
        for j in range(HALF // 16):
            rows0_v[r, pl.ds(j * 16, 16)] = jnp.zeros((16,), jnp.float32)
        return carry
    lax.fori_loop(0, ZROWS, zrow, None)
    for i in range(ROWS_PER_SUB // ZROWS):
        pltpu.sync_copy(rows0_v,
                        acc.at[pl.ds(sid * ROWS_PER_SUB + i * ZROWS, ZROWS)])
    t0 = sid * TAB_STRIPE
    pltpu.sync_copy(xt_hbm.at[cid, pl.ds(t0, TAB_STRIPE)],
                    tab.at[pl.ds(t0, TAB_STRIPE)])
    plsc.subcore_barrier()

    def scale(k, rbuf, sbuf):
        for g in range(CH // 16):
            wvec = wb_v[k, pl.ds(g * 16, 16)]
            for i in range(16):
                wb = lax.gather(
                    wvec, jnp.full((16, 1), i, jnp.int32), _GDN,
                    slice_sizes=(1,),
                    mode=lax.GatherScatterMode.PROMISE_IN_BOUNDS)
                e = g * 16 + i
                for j in range(HALF // 16):
                    sl = pl.ds(j * 16, 16)
                    sbuf[e, sl] = rbuf[e, sl] * wb

    gbufs = ((rows0_v, sem0), (rows1_v, sem1))
    sbufs = ((sbuf0_v, ssem0), (sbuf1_v, ssem1))

    def block(blk, carry):
        b0 = blk * BLK
        pltpu.sync_copy(src_hbm.at[sid, pl.ds(b0, BLK)], srcb_v)
        pltpu.sync_copy(dst_hbm.at[sid, pl.ds(b0, BLK)], dstb_v)
        pltpu.sync_copy(w_hbm.at[sid, pl.ds(b0, BLK)], wb_v)
        for b in range(2):
            rbuf, sem = gbufs[b]
            pltpu.make_async_copy(tab.at[srcb_v.at[b]], rbuf, sem).start()

        def pair(i, carry2):
            for b in range(2):
                kk = 2 * i + b
                rbuf, gsem = gbufs[b]
                sbuf, ssem = sbufs[b]

                @pl.when(kk >= 2)
                def _drain():
                    pltpu.make_async_copy(
                        sbuf, acc.at[dstb_v.at[kk]], ssem).wait()
                pltpu.make_async_copy(tab.at[srcb_v.at[kk]], rbuf, gsem).wait()
                scale(kk, rbuf, sbuf)
                pltpu.async_copy(sbuf, acc.at[dstb_v.at[kk]], ssem, add=True)

                @pl.when(kk + 2 < BLK)
                def _prefetch():
                    pltpu.make_async_copy(
                        tab.at[srcb_v.at[kk + 2]], rbuf, gsem).start()
            return carry2
        lax.fori_loop(0, BLK // 2, pair, None)
        for b in range(2):
            sbuf, ssem = sbufs[b]
            pltpu.make_async_copy(sbuf, acc.at[dstb_v.at[BLK - 2 + b]],
                                  ssem).wait()
        return carry
    lax.fori_loop(0, NCH // BLK, block, None)
    plsc.subcore_barrier()

    for i in range(ROWS_PER_SUB // ZROWS):
        r0 = sid * ROWS_PER_SUB + i * ZROWS
        pltpu.sync_copy(acc.at[pl.ds(r0, ZROWS)], out_hbm.at[cid, pl.ds(r0, ZROWS)])


def _run_sc_agg(xt2, src, dst, w):
    mesh = plsc.VectorSubcoreMesh(core_axis_name="c", subcore_axis_name="s")
    k = functools.partial(
        pl.kernel,
        mesh=mesh,
        compiler_params=pltpu.CompilerParams(use_tc_tiling_on_sc=False),
        out_type=jax.ShapeDtypeStruct((2, N_PAD, HALF), jnp.float32),
        scratch_types=[
            pltpu.VMEM((BLK, CH), jnp.int32),
            pltpu.VMEM((BLK, CH), jnp.int32),
            pltpu.VMEM((BLK, CH), jnp.float32),
            pltpu.VMEM((CH, HALF), jnp.float32),
            pltpu.VMEM((CH, HALF), jnp.float32),
            pltpu.VMEM((CH, HALF), jnp.float32),
            pltpu.VMEM((CH, HALF), jnp.float32),
            pltpu.VMEM_SHARED((N_NODES, HALF), jnp.float32),
            pltpu.VMEM_SHARED((N_PAD, HALF), jnp.float32),
            pltpu.SemaphoreType.DMA,
            pltpu.SemaphoreType.DMA,
            pltpu.SemaphoreType.DMA,
            pltpu.SemaphoreType.DMA,
        ],
    )(_sc_agg_body)
    return k(xt2, src, dst, w)


def kernel(x, edge_index, edge_weight, W, b):
    x = x.astype(jnp.float32)
    W = W.astype(jnp.float32)
    b2 = b.astype(jnp.float32).reshape(1, D)

    src0 = edge_index[0].astype(jnp.int32).reshape(NSUB, 20000)
    dst0 = edge_index[1].astype(jnp.int32).reshape(NSUB, 20000)
    w0 = edge_weight.astype(jnp.float32).reshape(NSUB, 20000)
    pad = EPT - 20000
    src0 = jnp.pad(src0, ((0, 0), (0, pad))).reshape(NSUB, NCH, CH)
    dst0 = jnp.pad(dst0, ((0, 0), (0, pad)),
                   constant_values=N_PAD - 1).reshape(NSUB, NCH, CH)
    w0 = jnp.pad(w0, ((0, 0), (0, pad))).reshape(NSUB, NCH, CH)

    xt2 = _run_prologue(x, W, b2)
    parts = _run_sc_agg(xt2, src0, dst0, w0)
    return _run_epilogue(parts[0, :N_NODES], parts[1, :N_NODES])
